# baseline (speedup 1.0000x reference)
import functools

import jax
import jax.numpy as jnp
from jax import lax
from jax.experimental import pallas as pl
from jax.experimental.pallas import tpu as pltpu
from jax.experimental.pallas import tpu_sc as plsc

N = 10000
E = 320000
HID = 64
CNTW = 16

NC, NS = 2, 16
NW = NC * NS
EPW = E // NW
CHUNK = 125
NCHUNK = EPW // CHUNK
NPAD = 10240
RPS = NPAD // NS
ZCH = 64
ZREP = RPS // ZCH
NBUF = 5
DELAY = 2

_mesh = plsc.VectorSubcoreMesh(core_axis_name="c", subcore_axis_name="s")
_sc_params = pltpu.CompilerParams(use_tc_tiling_on_sc=False)


def _mm_body(x_ref, w_ref, o_ref):
    o_ref[...] = jnp.dot(x_ref[...], w_ref[...],
                         preferred_element_type=jnp.float32)


def _matmul(x, w):
    return pl.pallas_call(
        _mm_body,
        out_shape=jax.ShapeDtypeStruct((x.shape[0], w.shape[1]), jnp.float32),
    )(x, w)


def _combine1_body(s_ref, c_ref, r_ref, b_ref, h_ref):
    cnt = jnp.maximum(c_ref[0, :N, 0:1] + c_ref[1, :N, 0:1], 1.0)
    agg = (s_ref[0, :N, :] + s_ref[1, :N, :]) / cnt
    h_ref[...] = jnp.maximum(agg + b_ref[...] + r_ref[...], 0.0)


def _combine1(S, C, r1, b1):
    return pl.pallas_call(
        _combine1_body,
        out_shape=jax.ShapeDtypeStruct((N, HID), jnp.float32),
    )(S, C, r1, b1)


def _combine2_body(t_ref, c_ref, r_ref, w_ref, b_ref, o_ref):
    cnt = jnp.maximum(c_ref[0, :N, 0:1] + c_ref[1, :N, 0:1], 1.0)
    agg = (t_ref[0, :N, :] + t_ref[1, :N, :]) / cnt
    o_ref[...] = (jnp.dot(agg, w_ref[...], preferred_element_type=jnp.float32)
                  + b_ref[...] + r_ref[...])


def _combine2(T, C, r2, W2_l, b2):
    return pl.pallas_call(
        _combine2_body,
        out_shape=jax.ShapeDtypeStruct((N, W2_l.shape[1]), jnp.float32),
    )(T, C, r2, W2_l, b2)


def _seg_body(with_counts, *refs):
    if with_counts:
        (p_hbm, src_hbm, dst_hbm, s_out, c_out, acc_sh, cacc_sh,
         src_v, dst_v, *rest) = refs
        bufs = rest[:NBUF]
        zb_v, ones_v, zc_v = rest[NBUF:NBUF + 3]
        gsems = rest[NBUF + 3:2 * NBUF + 3]
        ssems = rest[2 * NBUF + 3:]
    else:
        (p_hbm, src_hbm, dst_hbm, s_out, acc_sh,
         src_v, dst_v, *rest) = refs
        bufs = rest[:NBUF]
        zb_v = rest[NBUF]
        gsems = rest[NBUF + 1:2 * NBUF + 1]
        ssems = rest[2 * NBUF + 1:]

    c = lax.axis_index("c")
    s = lax.axis_index("s")
    wid = s * NC + c
    base = s * RPS

    pltpu.sync_copy(src_hbm.at[wid], src_v)
    pltpu.sync_copy(dst_hbm.at[wid], dst_v)
    for k in range(NBUF):
        pltpu.async_copy(p_hbm.at[src_v.at[k]], bufs[k], gsems[k])

    zeros16 = jnp.zeros((16,), jnp.float32)
    ones16 = jnp.ones((16,), jnp.float32)

    @pl.loop(0, ZCH)
    def _(r):
        @pl.loop(0, HID, step=16)
        def _(cc):
            zb_v[r, pl.ds(cc, 16)] = zeros16
        if with_counts:
            zc_v[r, pl.ds(0, 16)] = zeros16

    if with_counts:
        @pl.loop(0, CHUNK)
        def _(r):
            ones_v[r, pl.ds(0, 16)] = ones16

    @pl.loop(0, ZREP)
    def _(k):
        pltpu.sync_copy(zb_v, acc_sh.at[pl.ds(base + k * ZCH, ZCH)])
        if with_counts:
            pltpu.sync_copy(zc_v, cacc_sh.at[pl.ds(base + k * ZCH, ZCH)])

    plsc.subcore_barrier()

    def wait_gather(j, b):
        pltpu.make_async_copy(p_hbm.at[src_v.at[j]], bufs[b], gsems[b]).wait()

    def issue_scatter(j, b):
        pltpu.async_copy(bufs[b], acc_sh.at[dst_v.at[j]], ssems[b], add=True)
        if with_counts:
            pltpu.async_copy(ones_v, cacc_sh.at[dst_v.at[j]], ssems[b],
                             add=True)

    def wait_scatter(j, b):
        pltpu.make_async_copy(bufs[b], acc_sh.at[dst_v.at[j]], ssems[b]).wait()
        if with_counts:
            pltpu.make_async_copy(ones_v, cacc_sh.at[dst_v.at[j]],
                                  ssems[b]).wait()

    for k in range(DELAY):
        wait_gather(k, k)
        issue_scatter(k, k)

    @pl.loop(DELAY, NCHUNK - NBUF + DELAY, step=NBUF)
    def _(j0):
        for k in range(NBUF):
            jj = j0 + k
            b = (DELAY + k) % NBUF
            bre = k % NBUF
            wait_gather(jj, b)
            issue_scatter(jj, b)
            wait_scatter(jj - DELAY, bre)
            pltpu.async_copy(p_hbm.at[src_v.at[jj - DELAY + NBUF]],
                             bufs[bre], gsems[bre])

    for k in range(NBUF - DELAY):
        jj = NCHUNK - NBUF + DELAY + k
        wait_gather(jj, jj % NBUF)
        issue_scatter(jj, jj % NBUF)
        wait_scatter(jj - DELAY, (jj - DELAY) % NBUF)

    for k in range(DELAY):
        jj = NCHUNK - DELAY + k
        wait_scatter(jj, jj % NBUF)

    plsc.subcore_barrier()

    pltpu.sync_copy(acc_sh.at[pl.ds(base, RPS)],
                    s_out.at[c, pl.ds(base, RPS)])
    if with_counts:
        pltpu.sync_copy(cacc_sh.at[pl.ds(base, RPS)],
                        c_out.at[c, pl.ds(base, RPS)])


def _seg_sum_counts(p, src, dst):
    out_type = (jax.ShapeDtypeStruct((NC, NPAD, HID), jnp.float32),
                jax.ShapeDtypeStruct((NC, NPAD, CNTW), jnp.float32))
    scratch = (
        [pltpu.VMEM_SHARED((NPAD, HID), jnp.float32),
         pltpu.VMEM_SHARED((NPAD, CNTW), jnp.float32),
         pltpu.VMEM((NCHUNK, CHUNK), jnp.int32),
         pltpu.VMEM((NCHUNK, CHUNK), jnp.int32)]
        + [pltpu.VMEM((CHUNK, HID), jnp.float32) for _ in range(NBUF)]
        + [pltpu.VMEM((ZCH, HID), jnp.float32),
           pltpu.VMEM((CHUNK, CNTW), jnp.float32),
           pltpu.VMEM((ZCH, CNTW), jnp.float32)]
        + [pltpu.SemaphoreType.DMA for _ in range(2 * NBUF)]
    )
    fn = pl.kernel(functools.partial(_seg_body, True), out_type=out_type,
                   mesh=_mesh, scratch_types=scratch,
                   compiler_params=_sc_params)
    return fn(p, src, dst)


def _seg_sum(p, src, dst):
    out_type = jax.ShapeDtypeStruct((NC, NPAD, HID), jnp.float32)
    scratch = (
        [pltpu.VMEM_SHARED((NPAD, HID), jnp.float32),
         pltpu.VMEM((NCHUNK, CHUNK), jnp.int32),
         pltpu.VMEM((NCHUNK, CHUNK), jnp.int32)]
        + [pltpu.VMEM((CHUNK, HID), jnp.float32) for _ in range(NBUF)]
        + [pltpu.VMEM((ZCH, HID), jnp.float32)]
        + [pltpu.SemaphoreType.DMA for _ in range(2 * NBUF)]
    )
    fn = pl.kernel(functools.partial(_seg_body, False), out_type=out_type,
                   mesh=_mesh, scratch_types=scratch,
                   compiler_params=_sc_params)
    return fn(p, src, dst)


@jax.jit
def kernel(x, edge_index, W1_l, b1, W1_r, W2_l, b2, W2_r):
    src = edge_index[0].astype(jnp.int32).reshape(NW, NCHUNK, CHUNK)
    dst = edge_index[1].astype(jnp.int32).reshape(NW, NCHUNK, CHUNK)
    b1r = b1.reshape(1, -1)
    b2r = b2.reshape(1, -1)

    p = _matmul(x, W1_l)
    r1 = _matmul(x, W1_r)
    S, C = _seg_sum_counts(p, src, dst)
    h = _combine1(S, C, r1, b1r)
    r2 = _matmul(h, W2_r)
    T = _seg_sum(h, src, dst)
    return _combine2(T, C, r2, W2_l, b2r)


# speedup vs baseline: 1.0246x; 1.0246x over previous
import functools

import jax
import jax.numpy as jnp
from jax import lax
from jax.experimental import pallas as pl
from jax.experimental.pallas import tpu as pltpu
from jax.experimental.pallas import tpu_sc as plsc

N = 10000
E = 320000
HID = 64
CNTW = 16

NC, NS = 2, 16
NW = NC * NS
EPW = E // NW
CHUNK = 125
NCHUNK = EPW // CHUNK
NPAD = 10240
RPS = NPAD // NS
ZCH = 64
ZREP = RPS // ZCH
NBUF = 5
DELAY = 2

_mesh = plsc.VectorSubcoreMesh(core_axis_name="c", subcore_axis_name="s")
_sc_params = pltpu.CompilerParams(use_tc_tiling_on_sc=False)


def _mm_body(x_ref, w_ref, o_ref):
    o_ref[...] = jnp.dot(x_ref[...], w_ref[...],
                         preferred_element_type=jnp.float32)


def _matmul(x, w):
    return pl.pallas_call(
        _mm_body,
        out_shape=jax.ShapeDtypeStruct((x.shape[0], w.shape[1]), jnp.float32),
    )(x, w)


def _combine1_body(s_ref, c_ref, r_ref, b_ref, h_ref):
    cnt = jnp.maximum(c_ref[0, :N, 0:1] + c_ref[1, :N, 0:1], 1.0)
    agg = (s_ref[0, :N, :] + s_ref[1, :N, :]) / cnt
    h_ref[...] = jnp.maximum(agg + b_ref[...] + r_ref[...], 0.0)


def _combine1(S, C, r1, b1):
    return pl.pallas_call(
        _combine1_body,
        out_shape=jax.ShapeDtypeStruct((N, HID), jnp.float32),
    )(S, C, r1, b1)


def _combine2_body(t_ref, c_ref, r_ref, w_ref, b_ref, o_ref):
    cnt = jnp.maximum(c_ref[0, :N, 0:1] + c_ref[1, :N, 0:1], 1.0)
    agg = (t_ref[0, :N, :] + t_ref[1, :N, :]) / cnt
    o_ref[...] = (jnp.dot(agg, w_ref[...], preferred_element_type=jnp.float32)
                  + b_ref[...] + r_ref[...])


def _combine2(T, C, r2, W2_l, b2):
    return pl.pallas_call(
        _combine2_body,
        out_shape=jax.ShapeDtypeStruct((N, W2_l.shape[1]), jnp.float32),
    )(T, C, r2, W2_l, b2)


def _seg_body(with_counts, *refs):
    if with_counts:
        (p_hbm, src_hbm, dst_hbm, s_out, c_out, acc_sh, cacc_sh,
         src_v, dst_v, *rest) = refs
        bufs = rest[:NBUF]
        zb_v, ones_v, zc_v = rest[NBUF:NBUF + 3]
        gsems = rest[NBUF + 3:2 * NBUF + 3]
        ssems = rest[2 * NBUF + 3:]
    else:
        (p_hbm, src_hbm, dst_hbm, s_out, acc_sh,
         src_v, dst_v, *rest) = refs
        bufs = rest[:NBUF]
        zb_v = rest[NBUF]
        gsems = rest[NBUF + 1:2 * NBUF + 1]
        ssems = rest[2 * NBUF + 1:]

    c = lax.axis_index("c")
    s = lax.axis_index("s")
    wid = s * NC + c
    base = s * RPS

    dst_cp = pltpu.async_copy(dst_hbm.at[wid], dst_v, ssems[0])
    pltpu.sync_copy(src_hbm.at[wid], src_v)
    for k in range(NBUF):
        pltpu.async_copy(p_hbm.at[src_v.at[k]], bufs[k], gsems[k])

    zeros16 = jnp.zeros((16,), jnp.float32)
    ones16 = jnp.ones((16,), jnp.float32)

    @pl.loop(0, ZCH)
    def _(r):
        @pl.loop(0, HID, step=16)
        def _(cc):
            zb_v[r, pl.ds(cc, 16)] = zeros16
        if with_counts:
            zc_v[r, pl.ds(0, 16)] = zeros16

    if with_counts:
        @pl.loop(0, CHUNK)
        def _(r):
            ones_v[r, pl.ds(0, 16)] = ones16

    @pl.loop(0, ZREP)
    def _(k):
        pltpu.async_copy(zb_v, acc_sh.at[pl.ds(base + k * ZCH, ZCH)],
                         ssems[1])
        if with_counts:
            pltpu.async_copy(zc_v, cacc_sh.at[pl.ds(base + k * ZCH, ZCH)],
                             ssems[2])

    @pl.loop(0, ZREP)
    def _(k):
        pltpu.make_async_copy(zb_v, acc_sh.at[pl.ds(base + k * ZCH, ZCH)],
                              ssems[1]).wait()
        if with_counts:
            pltpu.make_async_copy(zc_v, cacc_sh.at[pl.ds(base + k * ZCH, ZCH)],
                                  ssems[2]).wait()
    dst_cp.wait()

    plsc.subcore_barrier()

    def wait_gather(j, b):
        pltpu.make_async_copy(p_hbm.at[src_v.at[j]], bufs[b], gsems[b]).wait()

    def issue_scatter(j, b):
        pltpu.async_copy(bufs[b], acc_sh.at[dst_v.at[j]], ssems[b], add=True)
        if with_counts:
            pltpu.async_copy(ones_v, cacc_sh.at[dst_v.at[j]], ssems[b],
                             add=True)

    def wait_scatter(j, b):
        pltpu.make_async_copy(bufs[b], acc_sh.at[dst_v.at[j]], ssems[b]).wait()
        if with_counts:
            pltpu.make_async_copy(ones_v, cacc_sh.at[dst_v.at[j]],
                                  ssems[b]).wait()

    @pl.loop(0, NCHUNK - NBUF, step=NBUF)
    def _(j):
        for k in range(NBUF):
            jj = j + k
            wait_gather(jj, k)
            issue_scatter(jj, k)
            wait_scatter(jj, k)
            pltpu.async_copy(p_hbm.at[src_v.at[jj + NBUF]],
                             bufs[k], gsems[k])

    for k in range(NBUF):
        jj = NCHUNK - NBUF + k
        wait_gather(jj, k)
        issue_scatter(jj, k)
        wait_scatter(jj, k)

    plsc.subcore_barrier()

    out_cp = pltpu.make_async_copy(acc_sh.at[pl.ds(base, RPS)],
                                   s_out.at[c, pl.ds(base, RPS)], ssems[0])
    out_cp.start()
    if with_counts:
        pltpu.sync_copy(cacc_sh.at[pl.ds(base, RPS)],
                        c_out.at[c, pl.ds(base, RPS)])
    out_cp.wait()


def _seg_sum_counts(p, src, dst):
    out_type = (jax.ShapeDtypeStruct((NC, NPAD, HID), jnp.float32),
                jax.ShapeDtypeStruct((NC, NPAD, CNTW), jnp.float32))
    scratch = (
        [pltpu.VMEM_SHARED((NPAD, HID), jnp.float32),
         pltpu.VMEM_SHARED((NPAD, CNTW), jnp.float32),
         pltpu.VMEM((NCHUNK, CHUNK), jnp.int32),
         pltpu.VMEM((NCHUNK, CHUNK), jnp.int32)]
        + [pltpu.VMEM((CHUNK, HID), jnp.float32) for _ in range(NBUF)]
        + [pltpu.VMEM((ZCH, HID), jnp.float32),
           pltpu.VMEM((CHUNK, CNTW), jnp.float32),
           pltpu.VMEM((ZCH, CNTW), jnp.float32)]
        + [pltpu.SemaphoreType.DMA for _ in range(2 * NBUF)]
    )
    fn = pl.kernel(functools.partial(_seg_body, True), out_type=out_type,
                   mesh=_mesh, scratch_types=scratch,
                   compiler_params=_sc_params)
    return fn(p, src, dst)


def _seg_sum(p, src, dst):
    out_type = jax.ShapeDtypeStruct((NC, NPAD, HID), jnp.float32)
    scratch = (
        [pltpu.VMEM_SHARED((NPAD, HID), jnp.float32),
         pltpu.VMEM((NCHUNK, CHUNK), jnp.int32),
         pltpu.VMEM((NCHUNK, CHUNK), jnp.int32)]
        + [pltpu.VMEM((CHUNK, HID), jnp.float32) for _ in range(NBUF)]
        + [pltpu.VMEM((ZCH, HID), jnp.float32)]
        + [pltpu.SemaphoreType.DMA for _ in range(2 * NBUF)]
    )
    fn = pl.kernel(functools.partial(_seg_body, False), out_type=out_type,
                   mesh=_mesh, scratch_types=scratch,
                   compiler_params=_sc_params)
    return fn(p, src, dst)


@jax.jit
def kernel(x, edge_index, W1_l, b1, W1_r, W2_l, b2, W2_r):
    src = edge_index[0].astype(jnp.int32).reshape(NW, NCHUNK, CHUNK)
    dst = edge_index[1].astype(jnp.int32).reshape(NW, NCHUNK, CHUNK)
    b1r = b1.reshape(1, -1)
    b2r = b2.reshape(1, -1)

    p = _matmul(x, W1_l)
    r1 = _matmul(x, W1_r)
    S, C = _seg_sum_counts(p, src, dst)
    h = _combine1(S, C, r1, b1r)
    r2 = _matmul(h, W2_r)
    T = _seg_sum(h, src, dst)
    return _combine2(T, C, r2, W2_l, b2r)
